# baseline (speedup 1.0000x reference)
import functools

import jax
import jax.numpy as jnp
from jax import lax
from jax.experimental import pallas as pl
from jax.experimental.pallas import tpu as pltpu
from jax.experimental.pallas import tpu_sc as plsc

VOCAB = 1000000
DIM = 64
ROWS = 16384
COLS = 50
NC, NS = 2, 16
NW = NC * NS
R_W = ROWS // NW
S_BLK = 128
N_S = R_W // S_BLK
N_UNITS = COLS * N_S
NT = S_BLK // 16

_MESH = plsc.VectorSubcoreMesh(
    core_axis_name="c", subcore_axis_name="s", num_cores=NC, num_subcores=NS
)


@functools.partial(
    pl.kernel,
    out_type=jax.ShapeDtypeStruct((COLS, DIM, ROWS), jnp.float32),
    mesh=_MESH,
    scratch_types=[
        pltpu.VMEM((R_W,), jnp.int32),
        pltpu.VMEM((2, R_W), jnp.int32),
        pltpu.VMEM((2, R_W), jnp.int32),
        pltpu.VMEM((N_S, S_BLK, 2 * DIM), jnp.float32),
        pltpu.VMEM((2, DIM, S_BLK), jnp.float32),
        pltpu.SemaphoreType.DMA((N_S,)),
        pltpu.SemaphoreType.DMA((2,)),
    ],
    compiler_params=pltpu.CompilerParams(needs_layout_passes=False),
)
def _gather_kernel(xt_hbm, w2_hbm, out_hbm, ibuf, idx2, parb, gbuf, obuf,
                   gsems, ssems):
    wid = lax.axis_index("s") * NC + lax.axis_index("c")
    r0 = wid * R_W

    iota = lax.iota(jnp.int32, 16)
    kvecs = [iota + (16 * t) for t in range(NT)]

    def load_row(c1, slot):
        pltpu.sync_copy(xt_hbm.at[c1, pl.ds(r0, R_W)], ibuf)
        for j in range(R_W // 16):
            v = ibuf[pl.ds(16 * j, 16)]
            idx2[slot, pl.ds(16 * j, 16)] = lax.shift_right_logical(v, 1)
            parb[slot, pl.ds(16 * j, 16)] = lax.bitwise_and(v, 1) * DIM

    def fire_g(s, slot, ring):
        pltpu.async_copy(
            w2_hbm.at[idx2.at[slot, pl.ds(s * S_BLK, S_BLK)]],
            gbuf.at[ring], gsems.at[ring],
        )

    def wait_g(ring):
        pltpu.make_async_copy(
            w2_hbm.at[pl.ds(0, S_BLK)], gbuf.at[ring], gsems.at[ring]
        ).wait()

    def fire_s(c, s, ob):
        pltpu.async_copy(
            obuf.at[ob], out_hbm.at[c, :, pl.ds(r0 + s * S_BLK, S_BLK)],
            ssems.at[ob],
        )

    def wait_s(ob):
        pltpu.make_async_copy(
            obuf.at[ob], out_hbm.at[0, :, pl.ds(0, S_BLK)], ssems.at[ob]
        ).wait()

    def select(s, slot, ring, ob):
        svec = jnp.full((16,), 0, jnp.int32) + ring
        cols = [
            parb[slot, pl.ds(s * S_BLK + 16 * t, 16)] for t in range(NT)
        ]
        for d in range(DIM):
            for t in range(NT):
                vals = plsc.load_gather(gbuf, [svec, kvecs[t], cols[t]])
                obuf[ob, d, pl.ds(16 * t, 16)] = vals
                cols[t] = cols[t] + 1

    load_row(0, 0)
    for s in range(N_S):
        fire_g(s, 0, s)

    def unit(u, _):
        c = u // N_S
        s = lax.rem(u, N_S)
        ring = lax.rem(u, N_S)
        ob = lax.rem(u, 2)
        slot = lax.rem(c, 2)

        wait_g(ring)

        @pl.when(u >= 2)
        def _():
            wait_s(ob)

        select(s, slot, ring, ob)
        fire_s(c, s, ob)

        @pl.when(jnp.logical_and(s == 0, c + 1 < COLS))
        def _():
            load_row(c + 1, 1 - slot)

        @pl.when(c + 1 < COLS)
        def _():
            fire_g(s, 1 - slot, ring)
        return 0

    lax.fori_loop(0, N_UNITS, unit, 0)

    wait_s(0)
    wait_s(1)


def kernel(x, weight):
    w2 = weight.reshape(VOCAB // 2, 2 * DIM)
    out = _gather_kernel(x.T.astype(jnp.int32), w2)
    return out.transpose(2, 0, 1)


# speedup vs baseline: 1.0375x; 1.0375x over previous
import functools

import jax
import jax.numpy as jnp
from jax import lax
from jax.experimental import pallas as pl
from jax.experimental.pallas import tpu as pltpu
from jax.experimental.pallas import tpu_sc as plsc

VOCAB = 1000000
DIM = 64
ROWS = 16384
COLS = 50
NC, NS = 2, 16
NW = NC * NS
R_W = ROWS // NW
S_BLK = 128
N_S = R_W // S_BLK
N_UNITS = COLS * N_S
NT = S_BLK // 16

_MESH = plsc.VectorSubcoreMesh(
    core_axis_name="c", subcore_axis_name="s", num_cores=NC, num_subcores=NS
)


@functools.partial(
    pl.kernel,
    out_type=jax.ShapeDtypeStruct((COLS, DIM, ROWS), jnp.float32),
    mesh=_MESH,
    scratch_types=[
        pltpu.VMEM((R_W,), jnp.int32),
        pltpu.VMEM((2, R_W), jnp.int32),
        pltpu.VMEM((2, R_W), jnp.int32),
        pltpu.VMEM((N_S, S_BLK, 2 * DIM), jnp.float32),
        pltpu.VMEM((2, DIM, S_BLK), jnp.float32),
        pltpu.SemaphoreType.DMA((N_S,)),
        pltpu.SemaphoreType.DMA((2,)),
    ],
    compiler_params=pltpu.CompilerParams(needs_layout_passes=False),
)
def _gather_kernel(xt_hbm, w2_hbm, out_hbm, ibuf, idx2, parb, gbuf, obuf,
                   gsems, ssems):
    wid = lax.axis_index("s") * NC + lax.axis_index("c")
    r0 = wid * R_W

    iota = lax.iota(jnp.int32, 16)
    kvecs = [iota + (16 * t) for t in range(NT)]

    def load_row(c1, slot):
        pltpu.sync_copy(xt_hbm.at[c1, pl.ds(r0, R_W)], ibuf)
        def prep(j, _):
            v = ibuf[pl.ds(16 * j, 16)]
            idx2[slot, pl.ds(16 * j, 16)] = lax.shift_right_logical(v, 1)
            parb[slot, pl.ds(16 * j, 16)] = lax.bitwise_and(v, 1) * DIM
            return 0
        lax.fori_loop(0, R_W // 16, prep, 0)

    def fire_g(s, slot, ring):
        pltpu.async_copy(
            w2_hbm.at[idx2.at[slot, pl.ds(s * S_BLK, S_BLK)]],
            gbuf.at[ring], gsems.at[ring],
        )

    def wait_g(ring):
        pltpu.make_async_copy(
            w2_hbm.at[pl.ds(0, S_BLK)], gbuf.at[ring], gsems.at[ring]
        ).wait()

    def fire_s(c, s, ob):
        pltpu.async_copy(
            obuf.at[ob], out_hbm.at[c, :, pl.ds(r0 + s * S_BLK, S_BLK)],
            ssems.at[ob],
        )

    def wait_s(ob):
        pltpu.make_async_copy(
            obuf.at[ob], out_hbm.at[0, :, pl.ds(0, S_BLK)], ssems.at[ob]
        ).wait()

    def select(s, slot, ring, ob):
        svec = jnp.full((16,), 0, jnp.int32) + ring
        colinit = tuple(
            parb[slot, pl.ds(s * S_BLK + 16 * t, 16)] for t in range(NT)
        )

        def drow(d, cols):
            new = []
            for t in range(NT):
                vals = plsc.load_gather(gbuf, [svec, kvecs[t], cols[t]])
                obuf[ob, d, pl.ds(16 * t, 16)] = vals
                new.append(cols[t] + 1)
            return tuple(new)

        lax.fori_loop(0, DIM, drow, colinit)

    load_row(0, 0)
    for s in range(N_S):
        fire_g(s, 0, s)

    def unit(u, _):
        c = u // N_S
        s = lax.rem(u, N_S)
        ring = lax.rem(u, N_S)
        ob = lax.rem(u, 2)
        slot = lax.rem(c, 2)

        wait_g(ring)

        @pl.when(u >= 2)
        def _():
            wait_s(ob)

        select(s, slot, ring, ob)
        fire_s(c, s, ob)

        @pl.when(jnp.logical_and(s == 0, c + 1 < COLS))
        def _():
            load_row(c + 1, 1 - slot)

        @pl.when(c + 1 < COLS)
        def _():
            fire_g(s, 1 - slot, ring)
        return 0

    lax.fori_loop(0, N_UNITS, unit, 0)

    wait_s(0)
    wait_s(1)


def kernel(x, weight):
    w2 = weight.reshape(VOCAB // 2, 2 * DIM)
    out = _gather_kernel(x.T.astype(jnp.int32), w2)
    return out.transpose(2, 0, 1)


# speedup vs baseline: 1.5117x; 1.4571x over previous
import functools

import jax
import jax.numpy as jnp
from jax import lax
from jax.experimental import pallas as pl
from jax.experimental.pallas import tpu as pltpu
from jax.experimental.pallas import tpu_sc as plsc

VOCAB = 1000000
DIM = 64
ROWS = 16384
COLS = 50
NC, NS = 2, 16
NW = NC * NS
ROWS_W = ROWS // NW
RCHUNK = 8
N_CHUNKS = ROWS_W // RCHUNK

_MESH = plsc.VectorSubcoreMesh(
    core_axis_name="c", subcore_axis_name="s", num_cores=NC, num_subcores=NS
)


@functools.partial(
    pl.kernel,
    out_type=jax.ShapeDtypeStruct((ROWS, COLS, DIM), jnp.float32),
    mesh=_MESH,
    scratch_types=[
        pltpu.VMEM((RCHUNK, COLS), jnp.int32),
        pltpu.VMEM((RCHUNK, COLS), jnp.int32),
        pltpu.VMEM((RCHUNK, COLS, DIM), jnp.float32),
        pltpu.VMEM((RCHUNK, COLS, DIM), jnp.float32),
        pltpu.SemaphoreType.DMA,
        pltpu.SemaphoreType.DMA,
        pltpu.SemaphoreType.DMA,
        pltpu.SemaphoreType.DMA,
    ],
    compiler_params=pltpu.CompilerParams(use_tc_tiling_on_sc=False),
)
def _gather_kernel(x_hbm, w_hbm, out_hbm, idx0, idx1, rows0, rows1,
                   gsem0, gsem1, ssem0, ssem1):
    wid = lax.axis_index("s") * NC + lax.axis_index("c")
    xrow0 = wid * ROWS_W

    def idx_load(g, idx_v):
        pltpu.sync_copy(x_hbm.at[pl.ds(xrow0 + g * RCHUNK, RCHUNK)], idx_v)

    def fire_g(idx_v, rows_v, sem):
        for r in range(RCHUNK):
            pltpu.async_copy(w_hbm.at[idx_v.at[r]], rows_v.at[r], sem)

    def wait_g(rows_v, sem):
        pltpu.make_async_copy(out_hbm.at[pl.ds(0, RCHUNK)], rows_v, sem).wait()

    def fire_s(g, rows_v, sem):
        pltpu.async_copy(
            rows_v, out_hbm.at[pl.ds(xrow0 + g * RCHUNK, RCHUNK)], sem
        )

    def wait_s(rows_v, sem):
        pltpu.make_async_copy(rows_v, out_hbm.at[pl.ds(0, RCHUNK)], sem).wait()

    idx_load(0, idx0)
    fire_g(idx0, rows0, gsem0)

    def pair(i, _):
        j = i * 2

        @pl.when(i > 0)
        def _():
            wait_s(rows1, ssem1)

        idx_load(j + 1, idx1)
        fire_g(idx1, rows1, gsem1)

        wait_g(rows0, gsem0)
        fire_s(j, rows0, ssem0)

        @pl.when(j + 2 < N_CHUNKS)
        def _():
            wait_s(rows0, ssem0)
            idx_load(j + 2, idx0)
            fire_g(idx0, rows0, gsem0)

        wait_g(rows1, gsem1)
        fire_s(j + 1, rows1, ssem1)
        return 0

    lax.fori_loop(0, N_CHUNKS // 2, pair, 0)

    wait_s(rows0, ssem0)
    wait_s(rows1, ssem1)


def kernel(x, weight):
    return _gather_kernel(x.astype(jnp.int32), weight)


# speedup vs baseline: 1.5229x; 1.0074x over previous
import functools

import jax
import jax.numpy as jnp
from jax import lax
from jax.experimental import pallas as pl
from jax.experimental.pallas import tpu as pltpu
from jax.experimental.pallas import tpu_sc as plsc

VOCAB = 1000000
DIM = 64
ROWS = 16384
COLS = 50
N = ROWS * COLS
NC, NS = 2, 16
NW = NC * NS
PER_W = N // NW
IDX_W = 128
SUB = 4
CHUNK = IDX_W * SUB
N_CHUNKS = PER_W // CHUNK

_MESH = plsc.VectorSubcoreMesh(
    core_axis_name="c", subcore_axis_name="s", num_cores=NC, num_subcores=NS
)


@functools.partial(
    pl.kernel,
    out_type=jax.ShapeDtypeStruct((N, DIM), jnp.float32),
    mesh=_MESH,
    scratch_types=[
        pltpu.VMEM((SUB, IDX_W), jnp.int32),
        pltpu.VMEM((SUB, IDX_W), jnp.int32),
        pltpu.VMEM((CHUNK, DIM), jnp.float32),
        pltpu.VMEM((CHUNK, DIM), jnp.float32),
        pltpu.SemaphoreType.DMA,
        pltpu.SemaphoreType.DMA,
        pltpu.SemaphoreType.DMA,
        pltpu.SemaphoreType.DMA,
    ],
    compiler_params=pltpu.CompilerParams(use_tc_tiling_on_sc=False),
)
def _gather_kernel(x_hbm, w_hbm, out_hbm, idx0, idx1, rows0, rows1,
                   gsem0, gsem1, ssem0, ssem1):
    wid = lax.axis_index("s") * NC + lax.axis_index("c")
    row0 = wid * (PER_W // IDX_W)
    out0 = wid * PER_W

    def idx_load(g, idx_v):
        pltpu.sync_copy(x_hbm.at[pl.ds(row0 + g * SUB, SUB)], idx_v)

    def fire_g(idx_v, rows_v, sem):
        for b in range(SUB):
            pltpu.async_copy(
                w_hbm.at[idx_v.at[b]],
                rows_v.at[pl.ds(b * IDX_W, IDX_W)],
                sem,
            )

    def wait_g(rows_v, sem):
        pltpu.make_async_copy(w_hbm.at[pl.ds(0, CHUNK)], rows_v, sem).wait()

    def fire_s(g, rows_v, sem):
        pltpu.async_copy(rows_v, out_hbm.at[pl.ds(out0 + g * CHUNK, CHUNK)], sem)

    def wait_s(rows_v, sem):
        pltpu.make_async_copy(rows_v, out_hbm.at[pl.ds(0, CHUNK)], sem).wait()

    idx_load(0, idx0)
    fire_g(idx0, rows0, gsem0)

    def pair(i, _):
        j = i * 2

        @pl.when(i > 0)
        def _():
            wait_s(rows1, ssem1)

        idx_load(j + 1, idx1)
        fire_g(idx1, rows1, gsem1)

        wait_g(rows0, gsem0)
        fire_s(j, rows0, ssem0)

        @pl.when(j + 2 < N_CHUNKS)
        def _():
            wait_s(rows0, ssem0)
            idx_load(j + 2, idx0)
            fire_g(idx0, rows0, gsem0)

        wait_g(rows1, gsem1)
        fire_s(j + 1, rows1, ssem1)
        return 0

    lax.fori_loop(0, N_CHUNKS // 2, pair, 0)

    wait_s(rows0, ssem0)
    wait_s(rows1, ssem1)


def kernel(x, weight):
    x2d = x.reshape(N // IDX_W, IDX_W).astype(jnp.int32)
    out = _gather_kernel(x2d, weight)
    return out.reshape(ROWS, COLS, DIM)
